# baseline (speedup 1.0000x reference)
import functools

import jax
import jax.numpy as jnp
from jax.experimental import pallas as pl
from jax.experimental.pallas import tpu as pltpu

D_MODEL = 768
D_FF = 3072
N_EXP = 8
TOPK = 2
N_HEADS = 12
EPS = 1e-5
F_BLK = 768


def _ln(x, g, b):
    m = x.mean(-1, keepdims=True)
    v = ((x - m) ** 2).mean(-1, keepdims=True)
    return (x - m) / jnp.sqrt(v + EPS) * g + b


def _mha(x, lp):
    Bq, T, D = x.shape
    H = N_HEADS
    hd = D // H
    q = (x @ lp['wq'] + lp['bq']).reshape(Bq, T, H, hd).transpose(0, 2, 1, 3)
    k = (x @ lp['wk'] + lp['bk']).reshape(Bq, T, H, hd).transpose(0, 2, 1, 3)
    v = (x @ lp['wv'] + lp['bv']).reshape(Bq, T, H, hd).transpose(0, 2, 1, 3)
    s = jnp.einsum('bhtd,bhsd->bhts', q, k) / jnp.sqrt(jnp.float32(hd))
    a = jax.nn.softmax(s, axis=-1)
    o = jnp.einsum('bhts,bhsd->bhtd', a, v).transpose(0, 2, 1, 3).reshape(Bq, T, D)
    return o @ lp['wo'] + lp['bo']


R_BLK = 256


def _gmm_body(bids_ref, eids_ref, valids_ref, offs_ref,
              xs_ref, gs_ref, w1_ref, b1_ref, w2_ref, b2_ref, out_ref):
    s = pl.program_id(0)
    bid = bids_ref[s]
    eid = eids_ref[s]
    prev_bid = bids_ref[jnp.maximum(s - 1, 0)]
    first = (s == 0) | (bid != prev_bid)

    @pl.when(first)
    def _init():
        out_ref[...] = jnp.zeros_like(out_ref)

    @pl.when(valids_ref[s] > 0)
    def _compute():
        rows = bid * R_BLK + jax.lax.broadcasted_iota(jnp.int32, (R_BLK, 1), 0)
        mask = (rows >= offs_ref[eid]) & (rows < offs_ref[eid + 1])
        x = xs_ref[...]
        h = jnp.dot(x, w1_ref[0], preferred_element_type=jnp.float32)
        h = h + b1_ref[0, 0]
        h = 0.5 * h * (1.0 + jax.lax.erf(h * 0.7071067811865476))
        o = jnp.dot(h, w2_ref[0], preferred_element_type=jnp.float32)
        o = (o + b2_ref[0, 0]) * gs_ref[...]
        out_ref[...] += jnp.where(mask, o, 0.0)


def _moe(x, lp):
    T = x.shape[0]
    P = TOPK * T
    NB = P // R_BLK
    G = NB + N_EXP - 1

    logits = x @ lp['router']
    probs = jax.nn.softmax(logits, axis=-1)
    topk_probs, topk_idx = jax.lax.top_k(probs, TOPK)
    topk_probs = topk_probs / topk_probs.sum(-1, keepdims=True)

    e_flat = topk_idx.reshape(P)
    g_flat = topk_probs.reshape(P)
    order = jnp.argsort(e_flat)
    t_sorted = (order // TOPK).astype(jnp.int32)
    g_sorted = g_flat[order]

    counts = jnp.bincount(e_flat, length=N_EXP)
    offs = jnp.concatenate([jnp.zeros((1,), jnp.int32),
                            jnp.cumsum(counts).astype(jnp.int32)])
    first = offs[:N_EXP] // R_BLK
    last = jnp.maximum(offs[1:] - 1, 0) // R_BLK
    nvis = jnp.where(counts > 0, last - first + 1, 0)
    cum = jnp.cumsum(nvis)
    sids = jnp.arange(G)
    eids = jnp.clip(jnp.searchsorted(cum, sids, side='right'), 0, N_EXP - 1)
    eids = eids.astype(jnp.int32)
    valids = (sids < cum[-1]).astype(jnp.int32)
    vstart = cum[eids] - nvis[eids]
    bids = jnp.clip(first[eids] + sids - vstart, 0, NB - 1).astype(jnp.int32)

    xs = x[t_sorted]
    gs = g_sorted.reshape(P, 1)

    o_pairs = pl.pallas_call(
        _gmm_body,
        grid_spec=pltpu.PrefetchScalarGridSpec(
            num_scalar_prefetch=4,
            grid=(G,),
            in_specs=[
                pl.BlockSpec((R_BLK, D_MODEL), lambda s, b, e, v, o: (b[s], 0)),
                pl.BlockSpec((R_BLK, 1), lambda s, b, e, v, o: (b[s], 0)),
                pl.BlockSpec((1, D_MODEL, D_FF), lambda s, b, e, v, o: (e[s], 0, 0)),
                pl.BlockSpec((1, 1, D_FF), lambda s, b, e, v, o: (e[s], 0, 0)),
                pl.BlockSpec((1, D_FF, D_MODEL), lambda s, b, e, v, o: (e[s], 0, 0)),
                pl.BlockSpec((1, 1, D_MODEL), lambda s, b, e, v, o: (e[s], 0, 0)),
            ],
            out_specs=pl.BlockSpec((R_BLK, D_MODEL), lambda s, b, e, v, o: (b[s], 0)),
        ),
        out_shape=jax.ShapeDtypeStruct((P, D_MODEL), x.dtype),
    )(bids, eids, valids, offs,
      xs, gs, lp['w1'], lp['b1'].reshape(N_EXP, 1, D_FF),
      lp['w2'], lp['b2'].reshape(N_EXP, 1, D_MODEL))

    return jnp.zeros((T, D_MODEL), x.dtype).at[t_sorted].add(o_pairs)


def _forward(particles, action, domain_id, params):
    a = _ln(action @ params['ap_w'] + params['ap_b'], params['ap_g'], params['ap_be'])
    x = particles + a[:, None, :]
    x = x + params['dom'][domain_id][:, None, :]
    for lp in params['layers']:
        xn = _ln(x, lp['g1'], lp['b1n'])
        x = x + _mha(xn, lp)
        xn = _ln(x, lp['g2'], lp['b2n'])
        Bq, T, D = x.shape
        x = x + _moe(xn.reshape(Bq * T, D), lp).reshape(Bq, T, D)
    out = _ln(x, params['out_g'], params['out_bn'])
    return out @ params['op_w'] + params['op_b']


def kernel(particles, action, domain_id, params):
    return _forward(particles, action, domain_id, params)


# speedup vs baseline: 1.0103x; 1.0103x over previous
import functools

import jax
import jax.numpy as jnp
from jax.experimental import pallas as pl
from jax.experimental.pallas import tpu as pltpu

D_MODEL = 768
D_FF = 3072
N_EXP = 8
TOPK = 2
N_HEADS = 12
EPS = 1e-5
F_BLK = 768


def _ln(x, g, b):
    m = x.mean(-1, keepdims=True)
    v = ((x - m) ** 2).mean(-1, keepdims=True)
    return (x - m) / jnp.sqrt(v + EPS) * g + b


def _mha(x, lp):
    Bq, T, D = x.shape
    H = N_HEADS
    hd = D // H
    q = (x @ lp['wq'] + lp['bq']).reshape(Bq, T, H, hd).transpose(0, 2, 1, 3)
    k = (x @ lp['wk'] + lp['bk']).reshape(Bq, T, H, hd).transpose(0, 2, 1, 3)
    v = (x @ lp['wv'] + lp['bv']).reshape(Bq, T, H, hd).transpose(0, 2, 1, 3)
    s = jnp.einsum('bhtd,bhsd->bhts', q, k) / jnp.sqrt(jnp.float32(hd))
    a = jax.nn.softmax(s, axis=-1)
    o = jnp.einsum('bhts,bhsd->bhtd', a, v).transpose(0, 2, 1, 3).reshape(Bq, T, D)
    return o @ lp['wo'] + lp['bo']


R_BLK = 256


def _gmm_body(bids_ref, eids_ref, valids_ref, offs_ref,
              xs_ref, gs_ref, w1_ref, b1_ref, w2_ref, b2_ref, out_ref):
    s = pl.program_id(0)
    bid = bids_ref[s]
    eid = eids_ref[s]
    prev_bid = bids_ref[jnp.maximum(s - 1, 0)]
    first = (s == 0) | (bid != prev_bid)

    @pl.when(first)
    def _init():
        out_ref[...] = jnp.zeros_like(out_ref)

    @pl.when(valids_ref[s] > 0)
    def _compute():
        rows = bid * R_BLK + jax.lax.broadcasted_iota(jnp.int32, (R_BLK, 1), 0)
        mask = (rows >= offs_ref[eid]) & (rows < offs_ref[eid + 1])
        x = xs_ref[...]
        h = jnp.dot(x, w1_ref[0], preferred_element_type=jnp.float32)
        h = h + b1_ref[0, 0]
        h = 0.5 * h * (1.0 + jax.lax.erf(h * 0.7071067811865476))
        o = jnp.dot(h, w2_ref[0], preferred_element_type=jnp.float32)
        o = (o + b2_ref[0, 0]) * gs_ref[...]
        out_ref[...] += jnp.where(mask, o, 0.0)


def _moe(x, lp):
    T = x.shape[0]
    P = TOPK * T
    NB = P // R_BLK
    G = NB + N_EXP - 1

    logits = x @ lp['router']
    probs = jax.nn.softmax(logits, axis=-1)
    topk_probs, topk_idx = jax.lax.top_k(probs, TOPK)
    topk_probs = topk_probs / topk_probs.sum(-1, keepdims=True)

    e_flat = topk_idx.reshape(P)
    g_flat = topk_probs.reshape(P)
    order = jnp.argsort(e_flat)
    t_sorted = (order // TOPK).astype(jnp.int32)
    g_sorted = g_flat[order]
    inv = jnp.zeros((P,), jnp.int32).at[order].set(jnp.arange(P, dtype=jnp.int32))

    counts = jnp.bincount(e_flat, length=N_EXP)
    offs = jnp.concatenate([jnp.zeros((1,), jnp.int32),
                            jnp.cumsum(counts).astype(jnp.int32)])
    first = offs[:N_EXP] // R_BLK
    last = jnp.maximum(offs[1:] - 1, 0) // R_BLK
    nvis = jnp.where(counts > 0, last - first + 1, 0)
    cum = jnp.cumsum(nvis)
    sids = jnp.arange(G)
    eids = jnp.clip(jnp.searchsorted(cum, sids, side='right'), 0, N_EXP - 1)
    eids = eids.astype(jnp.int32)
    valids = (sids < cum[-1]).astype(jnp.int32)
    vstart = cum[eids] - nvis[eids]
    bids = jnp.clip(first[eids] + sids - vstart, 0, NB - 1).astype(jnp.int32)

    xs = x[t_sorted]
    gs = g_sorted.reshape(P, 1)

    o_pairs = pl.pallas_call(
        _gmm_body,
        grid_spec=pltpu.PrefetchScalarGridSpec(
            num_scalar_prefetch=4,
            grid=(G,),
            in_specs=[
                pl.BlockSpec((R_BLK, D_MODEL), lambda s, b, e, v, o: (b[s], 0)),
                pl.BlockSpec((R_BLK, 1), lambda s, b, e, v, o: (b[s], 0)),
                pl.BlockSpec((1, D_MODEL, D_FF), lambda s, b, e, v, o: (e[s], 0, 0)),
                pl.BlockSpec((1, 1, D_FF), lambda s, b, e, v, o: (e[s], 0, 0)),
                pl.BlockSpec((1, D_FF, D_MODEL), lambda s, b, e, v, o: (e[s], 0, 0)),
                pl.BlockSpec((1, 1, D_MODEL), lambda s, b, e, v, o: (e[s], 0, 0)),
            ],
            out_specs=pl.BlockSpec((R_BLK, D_MODEL), lambda s, b, e, v, o: (b[s], 0)),
        ),
        out_shape=jax.ShapeDtypeStruct((P, D_MODEL), x.dtype),
    )(bids, eids, valids, offs,
      xs, gs, lp['w1'], lp['b1'].reshape(N_EXP, 1, D_FF),
      lp['w2'], lp['b2'].reshape(N_EXP, 1, D_MODEL))

    return o_pairs[inv.reshape(T, TOPK)].sum(1)


def _forward(particles, action, domain_id, params):
    a = _ln(action @ params['ap_w'] + params['ap_b'], params['ap_g'], params['ap_be'])
    x = particles + a[:, None, :]
    x = x + params['dom'][domain_id][:, None, :]
    for lp in params['layers']:
        xn = _ln(x, lp['g1'], lp['b1n'])
        x = x + _mha(xn, lp)
        xn = _ln(x, lp['g2'], lp['b2n'])
        Bq, T, D = x.shape
        x = x + _moe(xn.reshape(Bq * T, D), lp).reshape(Bq, T, D)
    out = _ln(x, params['out_g'], params['out_bn'])
    return out @ params['op_w'] + params['op_b']


def kernel(particles, action, domain_id, params):
    return _forward(particles, action, domain_id, params)
